# Initial kernel scaffold; baseline (speedup 1.0000x reference)
import jax
import jax.numpy as jnp
from jax.experimental import pallas as pl


def kernel(pc, params):
    raise NotImplementedError("write your pallas kernel here")


# speedup vs baseline: 4.4584x; 4.4584x over previous
import functools

import jax
import jax.numpy as jnp
from jax import lax
from jax.experimental import pallas as pl
from jax.experimental.pallas import tpu as pltpu
from jax.experimental.pallas import tpu_sc as plsc

EPS = 1e-8
BN_EPS = 1e-5
K_NS = 16
NWORKERS = 32


def _nrm_from_stats(st, n):
    mu = st[0:1, :] / n
    ex2 = st[1:2, :] / n
    inv = 1.0 / jnp.sqrt(ex2 - mu * mu + BN_EPS)
    return mu, inv


def _acc_stats(st_ref, y):
    @pl.when(pl.program_id(0) == 0)
    def _():
        st_ref[...] = jnp.zeros_like(st_ref)
    s = jnp.sum(y, axis=0, keepdims=True)
    s2 = jnp.sum(y * y, axis=0, keepdims=True)
    st_ref[0:2, :] += jnp.concatenate([s, s2], axis=0)


def _brow(b):
    return jnp.concatenate([b[None, :], jnp.zeros((7, b.shape[0]), jnp.float32)], 0)


def _dot(a, b):
    return jnp.dot(a.astype(jnp.bfloat16), b.astype(jnp.bfloat16),
                   preferred_element_type=jnp.float32)


def _linear(x, wt, st=None, bias=None, want_stats=False):
    R, Cin = x.shape
    Cout = wt.shape[1]
    chunk = min(4096, R)
    nrows = float(R)
    has_pre = st is not None
    has_bias = bias is not None

    def kfn(*refs):
        it = iter(refs)
        x_ref = next(it)
        w_ref = next(it)
        st_ref = next(it) if has_pre else None
        b_ref = next(it) if has_bias else None
        y_ref = next(it)
        so_ref = next(it) if want_stats else None
        xv = x_ref[...]
        if has_pre:
            mu, inv = _nrm_from_stats(st_ref[...], nrows)
            xv = jnp.maximum((xv - mu) * inv, 0.0)
        y = _dot(xv, w_ref[...])
        if has_bias:
            y = y + b_ref[0:1, :]
        y_ref[...] = y
        if want_stats:
            _acc_stats(so_ref, y)

    in_specs = [pl.BlockSpec((chunk, Cin), lambda i: (i, 0)),
                pl.BlockSpec((Cin, Cout), lambda i: (0, 0))]
    args = [x, wt]
    if has_pre:
        in_specs.append(pl.BlockSpec((8, Cin), lambda i: (0, 0)))
        args.append(st)
    if has_bias:
        in_specs.append(pl.BlockSpec((8, Cout), lambda i: (0, 0)))
        args.append(bias)
    out_shape = [jax.ShapeDtypeStruct((R, Cout), jnp.float32)]
    out_specs = [pl.BlockSpec((chunk, Cout), lambda i: (i, 0))]
    if want_stats:
        out_shape.append(jax.ShapeDtypeStruct((8, Cout), jnp.float32))
        out_specs.append(pl.BlockSpec((8, Cout), lambda i: (0, 0)))
    res = pl.pallas_call(kfn, grid=(R // chunk,), in_specs=in_specs,
                         out_specs=out_specs, out_shape=out_shape)(*args)
    return res if want_stats else res[0]


def _apply(y, st, res=None, mode="bn_relu"):
    R, C = y.shape
    chunk = min(4096, R)
    nrows = float(R)
    has_res = res is not None

    def kfn(y_ref, st_ref, *rest):
        if has_res:
            r_ref, o_ref = rest
        else:
            (o_ref,) = rest
        mu, inv = _nrm_from_stats(st_ref[...], nrows)
        z = (y_ref[...] - mu) * inv
        if mode == "bn_relu":
            o_ref[...] = jnp.maximum(z, 0.0)
        elif mode == "bn_add_relu":
            o_ref[...] = jnp.maximum(z + r_ref[...], 0.0)
        else:
            o_ref[...] = jnp.maximum(z, 0.0) + r_ref[...]

    in_specs = [pl.BlockSpec((chunk, C), lambda i: (i, 0)),
                pl.BlockSpec((8, C), lambda i: (0, 0))]
    args = [y, st]
    if has_res:
        in_specs.append(pl.BlockSpec((chunk, C), lambda i: (i, 0)))
        args.append(res)
    return pl.pallas_call(
        kfn, grid=(R // chunk,), in_specs=in_specs,
        out_specs=pl.BlockSpec((chunk, C), lambda i: (i, 0)),
        out_shape=jax.ShapeDtypeStruct((R, C), jnp.float32))(*args)


def _knn(qp, rp, k, want_d=False, want_coords=False):
    B, Nq, _ = qp.shape
    Nr = rp.shape[1]
    chunk = 128
    nchunk = Nq // chunk

    def kfn(q_ref, r_ref, rs_ref, *orefs):
        q = q_ref[...]
        r = r_ref[0]
        qs = jnp.sum(q * q, axis=1, keepdims=True)
        qr = lax.dot_general(q.astype(jnp.bfloat16), r.astype(jnp.bfloat16),
                             (((1,), (1,)), ((), ())),
                             preferred_element_type=jnp.float32)
        d = (qs - 2.0 * qr) + rs_ref[0]
        lane = lax.broadcasted_iota(jnp.int32, (chunk, Nr), 1)
        b = pl.program_id(0)
        idxs, dvs, ncs = [], [], []
        for _ in range(k):
            m = jnp.min(d, axis=1, keepdims=True)
            am = jnp.min(jnp.where(d == m, lane, Nr), axis=1, keepdims=True)
            idxs.append(am)
            dvs.append(m)
            if want_coords:
                oh = (lane == am).astype(jnp.float32)
                ncs.append(jnp.dot(oh, r, precision=lax.Precision.HIGHEST,
                                   preferred_element_type=jnp.float32))
            d = jnp.where(lane == am, jnp.float32(jnp.inf), d)
        orefs[0][...] = jnp.concatenate(idxs, axis=1) + b * Nr
        nxt = 1
        if want_d:
            orefs[nxt][...] = jnp.concatenate(dvs, axis=1)
            nxt += 1
        if want_coords:
            orefs[nxt][...] = jnp.concatenate(ncs, axis=1)

    in_specs = [
        pl.BlockSpec((chunk, 3), lambda b, j: (b * nchunk + j, 0)),
        pl.BlockSpec((1, Nr, 3), lambda b, j: (b, 0, 0)),
        pl.BlockSpec((1, 1, Nr), lambda b, j: (b, 0, 0)),
    ]
    out_shape = [jax.ShapeDtypeStruct((B * Nq, k), jnp.int32)]
    out_specs = [pl.BlockSpec((chunk, k), lambda b, j: (b * nchunk + j, 0))]
    if want_d:
        out_shape.append(jax.ShapeDtypeStruct((B * Nq, k), jnp.float32))
        out_specs.append(pl.BlockSpec((chunk, k), lambda b, j: (b * nchunk + j, 0)))
    if want_coords:
        out_shape.append(jax.ShapeDtypeStruct((B * Nq, 3 * k), jnp.float32))
        out_specs.append(pl.BlockSpec((chunk, 3 * k), lambda b, j: (b * nchunk + j, 0)))
    rsq = jnp.sum(rp * rp, axis=2).reshape(B, 1, Nr)
    res = pl.pallas_call(kfn, grid=(B, nchunk), in_specs=in_specs,
                         out_specs=out_specs,
                         out_shape=out_shape)(qp.reshape(B * Nq, 3), rp, rsq)
    return res[0] if len(res) == 1 else res


def _fps_kernel(px_ref, py_ref, pz_ref, ox_ref, oy_ref, oz_ref, *, n, m):
    px, py, pz = px_ref[...], py_ref[...], pz_ref[...]
    rows = n // 128
    fi = (lax.broadcasted_iota(jnp.int32, (rows, 128), 0) * 128
          + lax.broadcasted_iota(jnp.int32, (rows, 128), 1))
    mrows = m // 128
    mi = (lax.broadcasted_iota(jnp.int32, (mrows, 128), 0) * 128
          + lax.broadcasted_iota(jnp.int32, (mrows, 128), 1))

    def body(i, c):
        dist, li, ox, oy, oz = c
        sel = fi == li
        lx = jnp.sum(jnp.where(sel, px, 0.0))
        ly = jnp.sum(jnp.where(sel, py, 0.0))
        lz = jnp.sum(jnp.where(sel, pz, 0.0))
        rec = mi == i
        ox = jnp.where(rec, lx, ox)
        oy = jnp.where(rec, ly, oy)
        oz = jnp.where(rec, lz, oz)
        d = (px - lx) ** 2 + (py - ly) ** 2 + (pz - lz) ** 2
        dist = jnp.minimum(dist, d)
        mx = jnp.max(dist)
        ni = jnp.min(jnp.where(dist == mx, fi, n))
        return dist, ni, ox, oy, oz

    dist0 = jnp.full((rows, 128), 1e10, jnp.float32)
    z = jnp.zeros((mrows, 128), jnp.float32)
    _, _, ox, oy, oz = lax.fori_loop(
        0, m, body, (dist0, jnp.array(0, jnp.int32), z, z, z))
    ox_ref[...] = ox
    oy_ref[...] = oy
    oz_ref[...] = oz


def _fps(p, m):
    B, n, _ = p.shape
    outs = []
    for b in range(B):
        px = p[b, :, 0].reshape(n // 128, 128)
        py = p[b, :, 1].reshape(n // 128, 128)
        pz = p[b, :, 2].reshape(n // 128, 128)
        ox, oy, oz = pl.pallas_call(
            functools.partial(_fps_kernel, n=n, m=m),
            out_shape=[jax.ShapeDtypeStruct((m // 128, 128), jnp.float32)] * 3,
        )(px, py, pz)
        outs.append(jnp.stack([ox.reshape(m), oy.reshape(m), oz.reshape(m)], -1))
    return jnp.stack(outs)


def _gather_rows(table, idx):
    R = idx.shape[0]
    D = table.shape[1]
    bpw = R // NWORKERS
    ch = 128
    nch = bpw // ch
    mesh = plsc.VectorSubcoreMesh(core_axis_name="c", subcore_axis_name="s")

    @functools.partial(
        pl.kernel, mesh=mesh,
        out_type=jax.ShapeDtypeStruct((R, D), jnp.float32),
        scratch_types=[pltpu.VMEM((bpw,), jnp.int32),
                       pltpu.VMEM((ch, D), jnp.float32),
                       pltpu.SemaphoreType.DMA])
    def gk(table_hbm, idx_hbm, out_hbm, idx_v, rows_v, sem):
        wid = lax.axis_index("s") * 2 + lax.axis_index("c")
        base = wid * bpw
        pltpu.sync_copy(idx_hbm.at[pl.ds(base, bpw)], idx_v)
        for c in range(nch):
            pltpu.async_copy(
                table_hbm.at[idx_v.at[pl.ds(c * ch, ch)]], rows_v, sem).wait()
            pltpu.sync_copy(rows_v, out_hbm.at[pl.ds(base + c * ch, ch)])

    return gk(table, idx)


def _pt_t1(npg, pts, wt):
    Rn = npg.shape[0]
    chunk = 4096

    def kfn(g_ref, p_ref, w_ref, o_ref, st_ref):
        g = g_ref[...]
        pc = p_ref[...]
        ctr = jnp.broadcast_to(pc[:, None, :], (chunk // 16, 16, 3)).reshape(chunk, 3)
        t1 = _dot(g - ctr, w_ref[...])
        o_ref[...] = t1
        _acc_stats(st_ref, t1)

    return pl.pallas_call(
        kfn, grid=(Rn // chunk,),
        in_specs=[pl.BlockSpec((chunk, 3), lambda i: (i, 0)),
                  pl.BlockSpec((chunk // 16, 3), lambda i: (i, 0)),
                  pl.BlockSpec((3, 3), lambda i: (0, 0))],
        out_specs=[pl.BlockSpec((chunk, 3), lambda i: (i, 0)),
                   pl.BlockSpec((8, 3), lambda i: (0, 0))],
        out_shape=[jax.ShapeDtypeStruct((Rn, 3), jnp.float32),
                   jax.ShapeDtypeStruct((8, 3), jnp.float32)])(npg, pts, wt)


def _pt_a0(q, nkv, t1, st1, pe2t, pe2b):
    Rn = nkv.shape[0]
    chunk = 4096
    n1 = float(Rn)

    def kfn(q_ref, nk_ref, t1_ref, st1_ref, w_ref, b_ref, o_ref, st_ref):
        mu, inv = _nrm_from_stats(st1_ref[...], n1)
        r = jnp.maximum((t1_ref[...] - mu) * inv, 0.0)
        nr = _dot(r, w_ref[...]) + b_ref[0:1, :]
        qv = q_ref[...]
        qb = jnp.broadcast_to(qv[:, None, :], (chunk // 16, 16, 128)).reshape(chunk, 128)
        a0 = qb - nk_ref[...][:, 0:128] + nr
        o_ref[...] = a0
        _acc_stats(st_ref, a0)

    return pl.pallas_call(
        kfn, grid=(Rn // chunk,),
        in_specs=[pl.BlockSpec((chunk // 16, 128), lambda i: (i, 0)),
                  pl.BlockSpec((chunk, 256), lambda i: (i, 0)),
                  pl.BlockSpec((chunk, 3), lambda i: (i, 0)),
                  pl.BlockSpec((8, 3), lambda i: (0, 0)),
                  pl.BlockSpec((3, 128), lambda i: (0, 0)),
                  pl.BlockSpec((8, 128), lambda i: (0, 0))],
        out_specs=[pl.BlockSpec((chunk, 128), lambda i: (i, 0)),
                   pl.BlockSpec((8, 128), lambda i: (0, 0))],
        out_shape=[jax.ShapeDtypeStruct((Rn, 128), jnp.float32),
                   jax.ShapeDtypeStruct((8, 128), jnp.float32)])(
            q, nkv, t1, st1, pe2t, pe2b)


def _pt_attn(t2, st2, t1, st1, nkv, pe2t, pe2b, at2t, at2b):
    Rn = nkv.shape[0]
    chunk = 4096
    n1 = float(Rn)

    def kfn(t2_ref, st2_ref, t1_ref, st1_ref, nv_ref, wp_ref, bp_ref,
            wa_ref, ba_ref, o_ref, st_ref):
        mu1, inv1 = _nrm_from_stats(st1_ref[...], n1)
        r = jnp.maximum((t1_ref[...] - mu1) * inv1, 0.0)
        nr = _dot(r, wp_ref[...]) + bp_ref[0:1, :]
        mu2, inv2 = _nrm_from_stats(st2_ref[...], n1)
        h = jnp.maximum((t2_ref[...] - mu2) * inv2, 0.0)
        a = _dot(h, wa_ref[...]) + ba_ref[0:1, :]
        g = a.reshape(chunk // 16, 16, 128)
        gm = jnp.max(g, axis=1, keepdims=True)
        e = jnp.exp(g - gm)
        prob = e / jnp.sum(e, axis=1, keepdims=True)
        nv2 = (nv_ref[...][:, 128:256] + nr).reshape(chunk // 16, 16, 128)
        out = jnp.sum(nv2 * prob, axis=1)
        o_ref[...] = out
        _acc_stats(st_ref, out)

    Rp = Rn // 16
    return pl.pallas_call(
        kfn, grid=(Rn // chunk,),
        in_specs=[pl.BlockSpec((chunk, 128), lambda i: (i, 0)),
                  pl.BlockSpec((8, 128), lambda i: (0, 0)),
                  pl.BlockSpec((chunk, 3), lambda i: (i, 0)),
                  pl.BlockSpec((8, 3), lambda i: (0, 0)),
                  pl.BlockSpec((chunk, 256), lambda i: (i, 0)),
                  pl.BlockSpec((3, 128), lambda i: (0, 0)),
                  pl.BlockSpec((8, 128), lambda i: (0, 0)),
                  pl.BlockSpec((128, 128), lambda i: (0, 0)),
                  pl.BlockSpec((8, 128), lambda i: (0, 0))],
        out_specs=[pl.BlockSpec((chunk // 16, 128), lambda i: (i, 0)),
                   pl.BlockSpec((8, 128), lambda i: (0, 0))],
        out_shape=[jax.ShapeDtypeStruct((Rp, 128), jnp.float32),
                   jax.ShapeDtypeStruct((8, 128), jnp.float32)])(
            t2, st2, t1, st1, nkv, pe2t, pe2b, at2t, at2b)


def _pt_block(x, pts_flat, npg, idxf, w):
    t, st = _linear(x, w['lin1'].T, want_stats=True)
    q = _linear(t, w['wq'].T, st=st, bias=_brow(w['bq']))
    k = _linear(t, w['wk'].T, st=st, bias=_brow(w['bk']))
    v = _linear(t, w['wv'].T, st=st, bias=_brow(w['bv']))
    nkv = _gather_rows(jnp.concatenate([k, v], axis=1), idxf.reshape(-1))
    t1, st1 = _pt_t1(npg, pts_flat, w['pe1'].T)
    a0, st0 = _pt_a0(q, nkv, t1, st1, w['pe2'].T, _brow(w['pe2b']))
    t2, st2 = _linear(a0, w['at1'].T, st=st0, want_stats=True)
    po, stp = _pt_attn(t2, st2, t1, st1, nkv, w['pe2'].T, _brow(w['pe2b']),
                       w['at2'].T, _brow(w['at2b']))
    t3, st3 = _linear(po, w['lin2'].T, st=stp, want_stats=True)
    return _apply(t3, st3, res=x, mode="bn_add_relu")


def _down_t1(npg, nxf, ctr_pts, wt):
    Rn = npg.shape[0]
    chunk = 4096

    def kfn(g_ref, x_ref, p_ref, w_ref, o_ref, st_ref):
        g = g_ref[...]
        pc = p_ref[...]
        ctr = jnp.broadcast_to(pc[:, None, :], (chunk // 16, 16, 3)).reshape(chunk, 3)
        xin = jnp.concatenate([g - ctr, x_ref[...]], axis=1)
        t = _dot(xin, w_ref[...])
        o_ref[...] = t
        _acc_stats(st_ref, t)

    return pl.pallas_call(
        kfn, grid=(Rn // chunk,),
        in_specs=[pl.BlockSpec((chunk, 3), lambda i: (i, 0)),
                  pl.BlockSpec((chunk, 128), lambda i: (i, 0)),
                  pl.BlockSpec((chunk // 16, 3), lambda i: (i, 0)),
                  pl.BlockSpec((131, 128), lambda i: (0, 0))],
        out_specs=[pl.BlockSpec((chunk, 128), lambda i: (i, 0)),
                   pl.BlockSpec((8, 128), lambda i: (0, 0))],
        out_shape=[jax.ShapeDtypeStruct((Rn, 128), jnp.float32),
                   jax.ShapeDtypeStruct((8, 128), jnp.float32)])(
            npg, nxf, ctr_pts, wt)


def _down_max(t2, st2):
    Rn = t2.shape[0]
    chunk = 4096
    n2 = float(Rn)

    def kfn(t_ref, st_ref, o_ref):
        mu, inv = _nrm_from_stats(st_ref[...], n2)
        z = jnp.maximum((t_ref[...] - mu) * inv, 0.0)
        o_ref[...] = jnp.max(z.reshape(chunk // 16, 16, 128), axis=1)

    return pl.pallas_call(
        kfn, grid=(Rn // chunk,),
        in_specs=[pl.BlockSpec((chunk, 128), lambda i: (i, 0)),
                  pl.BlockSpec((8, 128), lambda i: (0, 0))],
        out_specs=pl.BlockSpec((chunk // 16, 128), lambda i: (i, 0)),
        out_shape=jax.ShapeDtypeStruct((Rn // 16, 128), jnp.float32))(t2, st2)


def _up_interp(g3, d3):
    Rp = d3.shape[0]
    chunk = 1024

    def kfn(g_ref, d_ref, o_ref):
        d = d_ref[...]
        dist = jnp.sqrt(jnp.maximum(d, 0.0))
        rec = 1.0 / (dist + EPS)
        wgt = rec / jnp.sum(rec, axis=1, keepdims=True)
        g = g_ref[...].reshape(chunk, 3, 128)
        out = (g[:, 0, :] * wgt[:, 0:1] + g[:, 1, :] * wgt[:, 1:2]
               + g[:, 2, :] * wgt[:, 2:3])
        o_ref[...] = out

    return pl.pallas_call(
        kfn, grid=(Rp // chunk,),
        in_specs=[pl.BlockSpec((chunk * 3, 128), lambda i: (i, 0)),
                  pl.BlockSpec((chunk, 3), lambda i: (i, 0))],
        out_specs=pl.BlockSpec((chunk, 128), lambda i: (i, 0)),
        out_shape=jax.ShapeDtypeStruct((Rp, 128), jnp.float32))(g3, d3)


def kernel(pc, params):
    B, N, _ = pc.shape
    M = N // 4
    p1 = pc[:, :, 0:3]
    p1f = p1.reshape(B * N, 3)
    x0 = pc[:, :, 3:].reshape(B * N, pc.shape[2] - 3)

    t, st = _linear(x0, params['in_w1'].T, want_stats=True)
    t, st = _linear(t, params['in_w2'].T, st=st, want_stats=True)
    x = _apply(t, st, mode="bn_relu")

    idx1, nc1 = _knn(p1, p1, K_NS, want_coords=True)
    np1 = nc1.reshape(B * N * K_NS, 3)
    x1 = _pt_block(x, p1f, np1, idx1, {k[3:]: v for k, v in params.items()
                                       if k.startswith('b1_')})

    p2 = _fps(p1, M)
    p2f = p2.reshape(B * M, 3)
    nidx, ncd = _knn(p2, p1, K_NS, want_coords=True)
    npd = ncd.reshape(B * M * K_NS, 3)
    nxd = _gather_rows(x1, nidx.reshape(-1))
    td, std = _down_t1(npd, nxd, p2f, params['dn1'].T)
    td2, std2 = _linear(td, params['dn2'].T, st=std, want_stats=True)
    x4 = _down_max(td2, std2)

    idx4, nc4 = _knn(p2, p2, K_NS, want_coords=True)
    np4 = nc4.reshape(B * M * K_NS, 3)
    x4 = _pt_block(x4, p2f, np4, idx4, {k[3:]: v for k, v in params.items()
                                        if k.startswith('b2_')})

    i3, d3 = _knn(p1, p2, 3, want_d=True)
    tu, stu = _linear(x4, params['up1'].T, want_stats=True)
    f1 = _apply(tu, stu, mode="bn_relu")
    g3 = _gather_rows(f1, i3.reshape(-1))
    upx = _up_interp(g3, d3)
    t2u, st2u = _linear(x1, params['up2'].T, want_stats=True)
    y = _apply(t2u, st2u, res=upx, mode="bn_relu_add")

    y = _pt_block(y, p1f, np1, idx1, {k[3:]: v for k, v in params.items()
                                      if k.startswith('b3_')})

    th, sth = _linear(y, params['out_w1'].T, want_stats=True)
    out = _linear(th, params['out_w2'].T, st=sth, bias=_brow(params['out_b2']))
    return out.reshape(B, N, -1).transpose(0, 2, 1)
